# baseline (speedup 1.0000x reference)
import jax
import jax.numpy as jnp
from jax.experimental import pallas as pl

B_BLK = 512
LOG2E = 1.4426950408889634
LN2 = 0.6931471805599453


def _loss_kernel(ev_ref, t_ref, beta_ref, gamma_ref, zbT_ref, zp_ref, out_ref):
    i = pl.program_id(0)

    t = t_ref[...]
    beta = beta_ref[...]
    gamma = gamma_ref[...]
    zbT = zbT_ref[...]
    zp = zp_ref[...]
    ev = ev_ref[...]

    zp0 = zp[0]
    zp1 = zp[1]
    zp2 = zp[2]

    S00 = jnp.sum(zp0 * zp0, axis=1, keepdims=True)
    S01 = jnp.sum(zp0 * zp1, axis=1, keepdims=True)
    Sm = jnp.sum(zp0 * zp2 + zp1 * zp1, axis=1, keepdims=True)
    S12 = jnp.sum(zp1 * zp2, axis=1, keepdims=True)
    S22 = jnp.sum(zp2 * zp2, axis=1, keepdims=True)

    v = jnp.concatenate([S00, S01, Sm, S12, S22, jnp.ones_like(S00),
                         zp0, zp1, zp2], axis=1)

    t2 = t * t
    nb = jnp.sum(zbT * zbT, axis=0, keepdims=True)
    u = jnp.concatenate(
        [jnp.ones_like(t), 2.0 * t, t2, t2 * t, 0.25 * (t2 * t2), nb,
         -2.0 * zbT, (-2.0 * t) * zbT, (-t2) * zbT], axis=0)

    d2 = jnp.dot(v, u, preferred_element_type=jnp.float32)
    dist = jnp.sqrt(jnp.maximum(d2, 0.0))
    arg = (gamma + beta) - dist

    sp = jnp.log2(1.0 + jnp.exp2(arg * LOG2E))
    s = (LN2 * jnp.sum(sp) - jnp.sum(jnp.where(ev != 0, arg, 0.0))).reshape(1, 1)

    @pl.when(i == 0)
    def _init():
        out_ref[...] = s

    @pl.when(i != 0)
    def _acc():
        out_ref[...] += s


def kernel(events, events_time, beta, gamma, z_b, z_p):
    N, B = events.shape
    O1, _, D = z_p.shape
    ev8 = events.astype(jnp.int8)
    t2d = events_time.reshape(1, B)
    beta2d = beta.reshape(1, B)
    gamma2d = gamma.reshape(N, 1)
    zbT = z_b.T

    nblk = B // B_BLK
    out = pl.pallas_call(
        _loss_kernel,
        grid=(nblk,),
        in_specs=[
            pl.BlockSpec((N, B_BLK), lambda i: (0, i)),
            pl.BlockSpec((1, B_BLK), lambda i: (0, i)),
            pl.BlockSpec((1, B_BLK), lambda i: (0, i)),
            pl.BlockSpec((N, 1), lambda i: (0, 0)),
            pl.BlockSpec((D, B_BLK), lambda i: (0, i)),
            pl.BlockSpec((O1, N, D), lambda i: (0, 0, 0)),
        ],
        out_specs=pl.BlockSpec((1, 1), lambda i: (0, 0)),
        out_shape=jax.ShapeDtypeStruct((1, 1), jnp.float32),
    )(ev8, t2d, beta2d, gamma2d, zbT, z_p)
    return out[0, 0]
